# baseline (speedup 1.0000x reference)
import functools

import jax
import jax.numpy as jnp
from jax import lax
from jax.experimental import pallas as pl
from jax.experimental.pallas import tpu as pltpu
from jax.experimental.pallas import tpu_sc as plsc

_B = 4 * 8192
_D = 1024
_INFO = plsc.get_sparse_core_info()
_NC = _INFO.num_cores
_NS = _INFO.num_subcores
_NW = _NC * _NS
_BPW = _B // _NW
_CHUNK = 16
_NB = 4
_N = _BPW // _CHUNK
_G = _N // _NB


@functools.partial(
    pl.kernel,
    mesh=plsc.VectorSubcoreMesh(core_axis_name="c", subcore_axis_name="s"),
    out_type=jax.ShapeDtypeStruct((_B, _D), jnp.float32),
    scratch_types=[
        pltpu.VMEM((_N, _CHUNK), jnp.int32),
        pltpu.VMEM((_NB, _CHUNK, _D), jnp.float32),
        pltpu.SemaphoreType.DMA((_NB,)),
        pltpu.SemaphoreType.DMA((_NB,)),
    ],
)
def _gather_rows(idx_hbm, table_hbm, out_hbm, idx_v, buf_v, gsem, ssem):
    wid = lax.axis_index("s") * _NC + lax.axis_index("c")
    base = wid * _BPW
    pltpu.sync_copy(idx_hbm.at[wid], idx_v)

    def start_gather(g, slot):
        pltpu.async_copy(table_hbm.at[idx_v.at[g]], buf_v.at[slot],
                         gsem.at[slot])

    def wait_gather(slot):
        pltpu.make_async_copy(table_hbm.at[idx_v.at[0]], buf_v.at[slot],
                              gsem.at[slot]).wait()

    def start_scatter(g, slot):
        pltpu.async_copy(buf_v.at[slot],
                         out_hbm.at[pl.ds(base + g * _CHUNK, _CHUNK)],
                         ssem.at[slot])

    def wait_scatter(slot):
        pltpu.make_async_copy(buf_v.at[slot],
                              out_hbm.at[pl.ds(base, _CHUNK)],
                              ssem.at[slot]).wait()

    def chunk_body(g, b, first, refill):
        prev = (b - 1) % _NB
        wait_gather(b)
        start_scatter(g, b)
        if not first:
            wait_scatter(prev)
        if refill:
            start_gather(g + _NB - 1, prev)

    for b in range(_NB - 1):
        start_gather(b, b)

    for b in range(_NB):
        chunk_body(b, b, first=(b == 0), refill=True)

    def group(i, carry):
        for b in range(_NB):
            chunk_body(i * _NB + b, b, first=False, refill=True)
        return carry

    lax.fori_loop(1, _G - 1, group, 0)

    for b in range(_NB):
        chunk_body(_N - _NB + b, b, first=False, refill=(b == 0))

    wait_scatter(_NB - 1)


def kernel(t, pos_emb):
    idx = t.reshape(_NW, _N, _CHUNK).astype(jnp.int32)
    out = _gather_rows(idx, pos_emb)
    return out.reshape(t.shape[0], t.shape[1], _D)
